# Initial kernel scaffold; baseline (speedup 1.0000x reference)
import jax
import jax.numpy as jnp
from jax.experimental import pallas as pl


def kernel(x, edge_index, W1, b1, W2, b2, W3, b3):
    raise NotImplementedError("write your pallas kernel here")


# speedup vs baseline: 442.1726x; 442.1726x over previous
import jax
import jax.numpy as jnp
from jax import lax
from jax.experimental import pallas as pl
from jax.experimental.pallas import tpu as pltpu
from jax.experimental.pallas import tpu_sc as plsc

_G, _N, _E = 2048, 64, 1024
_NC = 2
_NS = 16
_NW = _NC * _NS
_GPW = _G // _NW
_CHUNKS = _E // 16
_BG = 64
_M = float(_E + _N)


def _sc_hist_body(edges_hbm, out_hbm, edge_v, hist_v):
    wid = lax.axis_index("s") * _NC + lax.axis_index("c")
    base = wid * _GPW
    zeros16 = jnp.zeros((16,), jnp.float32)
    ones16 = jnp.ones((16,), jnp.float32)

    def zero_all(j, carry):
        hist_v[pl.ds(j * 16, 16)] = zeros16
        return carry

    lax.fori_loop(0, _N * _N // 16, zero_all, 0)

    def graph_body(i, carry):
        g = base + i
        pltpu.sync_copy(edges_hbm.at[g], edge_v)

        def acc_chunk(j, c2):
            r = edge_v[0, pl.ds(j * 16, 16)]
            c = edge_v[1, pl.ds(j * 16, 16)]
            plsc.addupdate_scatter(hist_v, [c * _N + r], ones16)
            return c2

        lax.fori_loop(0, _CHUNKS, acc_chunk, 0)
        pltpu.sync_copy(hist_v, out_hbm.at[g])

        def zero_chunk(j, c2):
            r = edge_v[0, pl.ds(j * 16, 16)]
            c = edge_v[1, pl.ds(j * 16, 16)]
            plsc.store_scatter(hist_v, [c * _N + r], zeros16)
            return c2

        lax.fori_loop(0, _CHUNKS, zero_chunk, 0)
        return carry

    lax.fori_loop(0, _GPW, graph_body, 0)


def _sc_hist(edge_index):
    mesh = plsc.VectorSubcoreMesh(core_axis_name="c", subcore_axis_name="s")
    return pl.kernel(
        _sc_hist_body,
        mesh=mesh,
        out_type=jax.ShapeDtypeStruct((_G, _N * _N), jnp.float32),
        scratch_types=[
            pltpu.VMEM((2, _E), jnp.int32),
            pltpu.VMEM((_N * _N,), jnp.float32),
        ],
        compiler_params=pltpu.CompilerParams(needs_layout_passes=False),
    )(edge_index)


def _tc_body(counts_ref, x_ref, w1_ref, b1_ref, w2_ref, b2_ref, w3_ref,
             b3_ref, out_ref):
    a = counts_ref[...]
    ii = lax.broadcasted_iota(jnp.int32, (_BG, _N, _N), 1)
    jj = lax.broadcasted_iota(jnp.int32, (_BG, _N, _N), 2)
    a = a + jnp.where(ii == jj, 1.0, 0.0)
    deg = jnp.sum(a, axis=2)
    dinv = lax.rsqrt(deg)
    av = jnp.sum(a * dinv[:, None, :], axis=2)
    s = jnp.sum(dinv * av, axis=1)

    def layer(h, w_ref, b_ref):
        t = lax.dot_general(h, w_ref[...], (((1,), (1,)), ((), ())),
                            preferred_element_type=jnp.float32)
        u = jnp.sum(a * t[:, None, :], axis=2)
        return s[:, None] * u + _M * b_ref[...]

    h1 = layer(x_ref[...], w1_ref, b1_ref)
    h2 = layer(h1, w2_ref, b2_ref)
    h3 = layer(h2, w3_ref, b3_ref)
    out_ref[...] = jnp.mean(h3, axis=1).reshape(1, 1, _BG)


def _tc_chain(counts, x, W1, b1, W2, b2, W3, b3):
    nblk = _G // _BG
    wspec = pl.BlockSpec((_N, _N), lambda i: (0, 0))
    bspec = pl.BlockSpec((1, _N), lambda i: (0, 0))
    out = pl.pallas_call(
        _tc_body,
        grid=(nblk,),
        in_specs=[
            pl.BlockSpec((_BG, _N, _N), lambda i: (i, 0, 0)),
            pl.BlockSpec((_BG, _N), lambda i: (i, 0)),
            wspec, bspec, wspec, bspec, wspec, bspec,
        ],
        out_specs=pl.BlockSpec((1, 1, _BG), lambda i: (i, 0, 0)),
        out_shape=jax.ShapeDtypeStruct((nblk, 1, _BG), jnp.float32),
    )(counts, x, W1, b1.reshape(1, _N), W2, b2.reshape(1, _N),
      W3, b3.reshape(1, _N))
    return out.reshape(_G)


def kernel(x, edge_index, W1, b1, W2, b2, W3, b3):
    counts = _sc_hist(edge_index).reshape(_G, _N, _N)
    return _tc_chain(counts, x, W1, b1, W2, b2, W3, b3)


# speedup vs baseline: 974.6156x; 2.2042x over previous
import jax
import jax.numpy as jnp
from jax import lax
from jax.experimental import pallas as pl
from jax.experimental.pallas import tpu as pltpu
from jax.experimental.pallas import tpu_sc as plsc

_G, _N, _E = 2048, 64, 1024
_NC = 2
_NS = 16
_NW = _NC * _NS
_GPW = _G // _NW
_CHUNKS = _E // 16
_UNROLL = 8
_NH = 4
_BG = _GPW
_M = float(_E + _N)


def _sc_hist_body(edges_hbm, out_hbm, e0, e1, e2, e3, e4, e5, e6, e7,
                  h0, h1, h2, h3, si0, si1, si2, si3, si4, si5, si6, si7,
                  so0, so1, so2, so3):
    ebufs = (e0, e1, e2, e3, e4, e5, e6, e7)
    hbufs = (h0, h1, h2, h3)
    sins = (si0, si1, si2, si3, si4, si5, si6, si7)
    souts = (so0, so1, so2, so3)
    wid = lax.axis_index("s") * _NC + lax.axis_index("c")
    base = wid * _GPW
    ones16 = jnp.ones((16,), jnp.float32)
    zeros16 = jnp.zeros((16,), jnp.float32)

    def zrow(j, carry):
        for hb in hbufs:
            hb[j, pl.ds(0, 16)] = zeros16
            hb[j, pl.ds(16, 16)] = zeros16
            hb[j, pl.ds(32, 16)] = zeros16
            hb[j, pl.ds(48, 16)] = zeros16
        return carry

    lax.fori_loop(0, _N, zrow, 0)

    for s in range(_UNROLL):
        pltpu.async_copy(edges_hbm.at[base + s], ebufs[s], sins[s])

    def pipe(k, carry):
        g0 = base + _UNROLL * k
        for s in range(_UNROLL):
            g = g0 + s
            eb, sin = ebufs[s], sins[s]
            hb, sout = hbufs[s % _NH], souts[s % _NH]
            pltpu.make_async_copy(edges_hbm.at[g], eb, sin).wait()
            if s < _NH:
                @pl.when(k > 0)
                def _():
                    pltpu.make_async_copy(hb, out_hbm.at[g - _NH], sout).wait()
            else:
                pltpu.make_async_copy(hb, out_hbm.at[g - _NH], sout).wait()
            @plsc.parallel_loop(0, _CHUNKS, 1, unroll=8)
            def _(j):
                r = eb[0, pl.ds(j * 16, 16)]
                c = eb[1, pl.ds(j * 16, 16)]
                plsc.addupdate_scatter(hb, [c, r], ones16)

            pltpu.async_copy(hb, out_hbm.at[g], sout)

            @pl.when(_UNROLL * k + s + _UNROLL < _GPW)
            def _():
                pltpu.async_copy(edges_hbm.at[g + _UNROLL], eb, sin)
        return carry

    lax.fori_loop(0, _GPW // _UNROLL, pipe, 0)

    for j in range(_NH):
        g = base + _GPW - _NH + j
        pltpu.make_async_copy(hbufs[j % _NH], out_hbm.at[g],
                              souts[j % _NH]).wait()


def _sc_hist(edge_index):
    mesh = plsc.VectorSubcoreMesh(core_axis_name="c", subcore_axis_name="s")
    return pl.kernel(
        _sc_hist_body,
        mesh=mesh,
        out_type=jax.ShapeDtypeStruct((_G, _N, _N), jnp.float32),
        scratch_types=(
            [pltpu.VMEM((2, _E), jnp.int32) for _ in range(_UNROLL)]
            + [pltpu.VMEM((_N, _N), jnp.float32) for _ in range(_NH)]
            + [pltpu.SemaphoreType.DMA for _ in range(_UNROLL + _NH)]
        ),
        compiler_params=pltpu.CompilerParams(needs_layout_passes=False),
    )(edge_index)


def _tc_body(counts_ref, x_ref, w1_ref, b1_ref, w2_ref, b2_ref, w3_ref,
             b3_ref, out_ref):
    acc = counts_ref[...]
    prev = jnp.concatenate(
        [jnp.zeros((_NH, _N, _N), jnp.float32), acc[:-_NH]], axis=0)
    a = acc - prev
    ii = lax.broadcasted_iota(jnp.int32, (_BG, _N, _N), 1)
    jj = lax.broadcasted_iota(jnp.int32, (_BG, _N, _N), 2)
    a = a + jnp.where(ii == jj, 1.0, 0.0)
    deg = jnp.sum(a, axis=2)
    dinv = lax.rsqrt(deg)
    av = jnp.sum(a * dinv[:, None, :], axis=2)
    s = jnp.sum(dinv * av, axis=1)

    def layer(h, w_ref, b_ref):
        t = lax.dot_general(h, w_ref[...], (((1,), (1,)), ((), ())),
                            preferred_element_type=jnp.float32)
        u = jnp.sum(a * t[:, None, :], axis=2)
        return s[:, None] * u + _M * b_ref[...]

    h1 = layer(x_ref[...], w1_ref, b1_ref)
    h2 = layer(h1, w2_ref, b2_ref)
    h3 = layer(h2, w3_ref, b3_ref)
    out_ref[...] = jnp.mean(h3, axis=1).reshape(1, 1, _BG)


def _tc_chain(counts, x, W1, b1, W2, b2, W3, b3):
    nblk = _G // _BG
    wspec = pl.BlockSpec((_N, _N), lambda i: (0, 0))
    bspec = pl.BlockSpec((1, _N), lambda i: (0, 0))
    out = pl.pallas_call(
        _tc_body,
        grid=(nblk,),
        in_specs=[
            pl.BlockSpec((_BG, _N, _N), lambda i: (i, 0, 0)),
            pl.BlockSpec((_BG, _N), lambda i: (i, 0)),
            wspec, bspec, wspec, bspec, wspec, bspec,
        ],
        out_specs=pl.BlockSpec((1, 1, _BG), lambda i: (i, 0, 0)),
        out_shape=jax.ShapeDtypeStruct((nblk, 1, _BG), jnp.float32),
    )(counts, x, W1, b1.reshape(1, _N), W2, b2.reshape(1, _N),
      W3, b3.reshape(1, _N))
    return out.reshape(_G)


def kernel(x, edge_index, W1, b1, W2, b2, W3, b3):
    counts = _sc_hist(edge_index)
    return _tc_chain(counts, x, W1, b1, W2, b2, W3, b3)


# speedup vs baseline: 1143.4217x; 1.1732x over previous
import jax
import jax.numpy as jnp
from jax import lax
from jax.experimental import pallas as pl
from jax.experimental.pallas import tpu as pltpu
from jax.experimental.pallas import tpu_sc as plsc

_G, _N, _E = 2048, 64, 1024
_NC = 2
_NS = 16
_NW = _NC * _NS
_GPW = _G // _NW
_CHUNKS = _E // 16
_UNROLL = 8
_NH = 4
_BG = _GPW
_M = float(_E + _N)


def _sc_hist_body(edges_hbm, cnt_hbm, dr_hbm, e0, e1, e2, e3, e4, e5, e6, e7,
                  h0, h1, h2, h3, d0, d1, d2, d3,
                  si0, si1, si2, si3, si4, si5, si6, si7,
                  so0, so1, so2, so3, sd0, sd1, sd2, sd3):
    ebufs = (e0, e1, e2, e3, e4, e5, e6, e7)
    hbufs = (h0, h1, h2, h3)
    dbufs = (d0, d1, d2, d3)
    sins = (si0, si1, si2, si3, si4, si5, si6, si7)
    souts = (so0, so1, so2, so3)
    sdrs = (sd0, sd1, sd2, sd3)
    wid = lax.axis_index("s") * _NC + lax.axis_index("c")
    base = wid * _GPW
    ones16 = jnp.ones((16,), jnp.float32)
    zeros16 = jnp.zeros((16,), jnp.float32)

    def zrow(j, carry):
        for hb in hbufs:
            hb[j, pl.ds(0, 16)] = zeros16
            hb[j, pl.ds(16, 16)] = zeros16
            hb[j, pl.ds(32, 16)] = zeros16
            hb[j, pl.ds(48, 16)] = zeros16
        return carry

    lax.fori_loop(0, _N, zrow, 0)
    for db in dbufs:
        for j in range(8):
            db[pl.ds(j * 16, 16)] = zeros16

    for s in range(_UNROLL):
        pltpu.async_copy(edges_hbm.at[base + s], ebufs[s], sins[s])

    def pipe(k, carry):
        g0 = base + _UNROLL * k
        for s in range(_UNROLL):
            g = g0 + s
            eb, sin = ebufs[s], sins[s]
            hb, sout = hbufs[s % _NH], souts[s % _NH]
            db, sdr = dbufs[s % _NH], sdrs[s % _NH]
            pltpu.make_async_copy(edges_hbm.at[g], eb, sin).wait()

            def _waits():
                pltpu.make_async_copy(hb, cnt_hbm.at[g - _NH], sout).wait()
                pltpu.make_async_copy(db, dr_hbm.at[g - _NH], sdr).wait()

            if s < _NH:
                pl.when(k > 0)(_waits)
            else:
                _waits()

            @plsc.parallel_loop(0, _CHUNKS, 1, unroll=8)
            def _(j):
                r = eb[0, pl.ds(j * 16, 16)]
                c = eb[1, pl.ds(j * 16, 16)]
                plsc.addupdate_scatter(hb, [r, c], ones16)
                plsc.addupdate_scatter(db, [c], ones16)
                plsc.addupdate_scatter(db, [r + _N], ones16)

            pltpu.async_copy(hb, cnt_hbm.at[g], sout)
            pltpu.async_copy(db, dr_hbm.at[g], sdr)

            @pl.when(_UNROLL * k + s + _UNROLL < _GPW)
            def _():
                pltpu.async_copy(edges_hbm.at[g + _UNROLL], eb, sin)
        return carry

    lax.fori_loop(0, _GPW // _UNROLL, pipe, 0)

    for j in range(_NH):
        g = base + _GPW - _NH + j
        pltpu.make_async_copy(hbufs[j], cnt_hbm.at[g], souts[j]).wait()
        pltpu.make_async_copy(dbufs[j], dr_hbm.at[g], sdrs[j]).wait()


def _sc_hist(edge_index):
    mesh = plsc.VectorSubcoreMesh(core_axis_name="c", subcore_axis_name="s")
    return pl.kernel(
        _sc_hist_body,
        mesh=mesh,
        out_type=(
            jax.ShapeDtypeStruct((_G, _N, _N), jnp.float32),
            jax.ShapeDtypeStruct((_G, 2 * _N), jnp.float32),
        ),
        scratch_types=(
            [pltpu.VMEM((2, _E), jnp.int32) for _ in range(_UNROLL)]
            + [pltpu.VMEM((_N, _N), jnp.float32) for _ in range(_NH)]
            + [pltpu.VMEM((2 * _N,), jnp.float32) for _ in range(_NH)]
            + [pltpu.SemaphoreType.DMA for _ in range(_UNROLL + 2 * _NH)]
        ),
        compiler_params=pltpu.CompilerParams(needs_layout_passes=False),
    )(edge_index)


def _tc_body(counts_ref, dr_ref, x_ref, w1_ref, b1_ref, w2_ref, b2_ref,
             w3_ref, b3_ref, out_ref):
    acc = counts_ref[...]
    cprev = jnp.concatenate(
        [jnp.zeros((_NH, _N, _N), jnp.float32), acc[:-_NH]], axis=0)
    c = acc - cprev
    dacc = dr_ref[...]
    dprev = jnp.concatenate(
        [jnp.zeros((_NH, 2 * _N), jnp.float32), dacc[:-_NH]], axis=0)
    dr = dacc - dprev
    deg = dr[:, :_N] + 1.0
    rdeg = dr[:, _N:] + 1.0
    dinv = lax.rsqrt(deg)

    def cmatvec(v):
        return jnp.sum(c * v[:, :, None], axis=1) + v

    av = cmatvec(dinv)
    s = jnp.sum(dinv * av, axis=1)

    def layer(h, w_ref, b_ref):
        t = lax.dot_general(h, w_ref[...], (((1,), (1,)), ((), ())),
                            preferred_element_type=jnp.float32)
        u = cmatvec(t)
        return s[:, None] * u + _M * b_ref[...]

    h1 = layer(x_ref[...], w1_ref, b1_ref)
    h2 = layer(h1, w2_ref, b2_ref)
    t3 = lax.dot_general(h2, w3_ref[...], (((1,), (1,)), ((), ())),
                         preferred_element_type=jnp.float32)
    tot = s * jnp.sum(rdeg * t3, axis=1) + _M * jnp.sum(b3_ref[...])
    out_ref[...] = (tot * (1.0 / _N)).reshape(1, 1, _BG)


def _tc_chain(counts, dr, x, W1, b1, W2, b2, W3, b3):
    nblk = _G // _BG
    wspec = pl.BlockSpec((_N, _N), lambda i: (0, 0))
    bspec = pl.BlockSpec((1, _N), lambda i: (0, 0))
    out = pl.pallas_call(
        _tc_body,
        grid=(nblk,),
        in_specs=[
            pl.BlockSpec((_BG, _N, _N), lambda i: (i, 0, 0)),
            pl.BlockSpec((_BG, 2 * _N), lambda i: (i, 0)),
            pl.BlockSpec((_BG, _N), lambda i: (i, 0)),
            wspec, bspec, wspec, bspec, wspec, bspec,
        ],
        out_specs=pl.BlockSpec((1, 1, _BG), lambda i: (i, 0, 0)),
        out_shape=jax.ShapeDtypeStruct((nblk, 1, _BG), jnp.float32),
    )(counts, dr, x, W1, b1.reshape(1, _N), W2, b2.reshape(1, _N),
      W3, b3.reshape(1, _N))
    return out.reshape(_G)


def kernel(x, edge_index, W1, b1, W2, b2, W3, b3):
    counts, dr = _sc_hist(edge_index)
    return _tc_chain(counts, dr, x, W1, b1, W2, b2, W3, b3)


# speedup vs baseline: 1243.8983x; 1.0879x over previous
import jax
import jax.numpy as jnp
from jax import lax
from jax.experimental import pallas as pl
from jax.experimental.pallas import tpu as pltpu
from jax.experimental.pallas import tpu_sc as plsc

_G, _N, _E = 2048, 64, 1024
_NC = 2
_NS = 16
_NW = _NC * _NS
_GPW = _G // _NW
_CHUNKS = _E // 16
_UNROLL = 8
_NH = 4
_BG = 2 * _GPW
_M = float(_E + _N)


def _sc_hist_body(edges_hbm, cnt_hbm, dr_hbm, e0, e1, e2, e3, e4, e5, e6, e7,
                  h0, h1, h2, h3, d0, d1, d2, d3,
                  si0, si1, si2, si3, si4, si5, si6, si7,
                  so0, so1, so2, so3, sd0, sd1, sd2, sd3):
    ebufs = (e0, e1, e2, e3, e4, e5, e6, e7)
    hbufs = (h0, h1, h2, h3)
    dbufs = (d0, d1, d2, d3)
    sins = (si0, si1, si2, si3, si4, si5, si6, si7)
    souts = (so0, so1, so2, so3)
    sdrs = (sd0, sd1, sd2, sd3)
    wid = lax.axis_index("s") * _NC + lax.axis_index("c")
    base = wid * _GPW
    ones16 = jnp.ones((16,), jnp.float32)
    zeros16 = jnp.zeros((16,), jnp.float32)

    def zrow(j, carry):
        for hb in hbufs:
            hb[j, pl.ds(0, 16)] = zeros16
            hb[j, pl.ds(16, 16)] = zeros16
            hb[j, pl.ds(32, 16)] = zeros16
            hb[j, pl.ds(48, 16)] = zeros16
        return carry

    lax.fori_loop(0, _N, zrow, 0)
    for db in dbufs:
        for j in range(8):
            db[pl.ds(j * 16, 16)] = zeros16

    for s in range(_UNROLL):
        pltpu.async_copy(edges_hbm.at[base + s], ebufs[s], sins[s])

    def pipe(k, carry):
        g0 = base + _UNROLL * k
        for s in range(_UNROLL):
            g = g0 + s
            eb, sin = ebufs[s], sins[s]
            hb, sout = hbufs[s % _NH], souts[s % _NH]
            db, sdr = dbufs[s % _NH], sdrs[s % _NH]
            pltpu.make_async_copy(edges_hbm.at[g], eb, sin).wait()

            def _waits():
                pltpu.make_async_copy(hb, cnt_hbm.at[g - _NH], sout).wait()
                pltpu.make_async_copy(db, dr_hbm.at[g - _NH], sdr).wait()

            if s < _NH:
                pl.when(k > 0)(_waits)
            else:
                _waits()

            @plsc.parallel_loop(0, _CHUNKS, 1, unroll=8)
            def _(j):
                r = eb[0, pl.ds(j * 16, 16)]
                c = eb[1, pl.ds(j * 16, 16)]
                plsc.addupdate_scatter(hb, [r, c], ones16)
                plsc.addupdate_scatter(db, [c], ones16)
                plsc.addupdate_scatter(db, [r + _N], ones16)

            pltpu.async_copy(hb, cnt_hbm.at[g], sout)
            pltpu.async_copy(db, dr_hbm.at[g], sdr)

            @pl.when(_UNROLL * k + s + _UNROLL < _GPW)
            def _():
                pltpu.async_copy(edges_hbm.at[g + _UNROLL], eb, sin)
        return carry

    lax.fori_loop(0, _GPW // _UNROLL, pipe, 0)

    for j in range(_NH):
        g = base + _GPW - _NH + j
        pltpu.make_async_copy(hbufs[j], cnt_hbm.at[g], souts[j]).wait()
        pltpu.make_async_copy(dbufs[j], dr_hbm.at[g], sdrs[j]).wait()


def _sc_hist(edge_index):
    mesh = plsc.VectorSubcoreMesh(core_axis_name="c", subcore_axis_name="s")
    return pl.kernel(
        _sc_hist_body,
        mesh=mesh,
        out_type=(
            jax.ShapeDtypeStruct((_G, _N, _N), jnp.float32),
            jax.ShapeDtypeStruct((_G, 2 * _N), jnp.float32),
        ),
        scratch_types=(
            [pltpu.VMEM((2, _E), jnp.int32) for _ in range(_UNROLL)]
            + [pltpu.VMEM((_N, _N), jnp.float32) for _ in range(_NH)]
            + [pltpu.VMEM((2 * _N,), jnp.float32) for _ in range(_NH)]
            + [pltpu.SemaphoreType.DMA for _ in range(_UNROLL + 2 * _NH)]
        ),
        compiler_params=pltpu.CompilerParams(needs_layout_passes=False),
    )(edge_index)


def _tc_body(counts_ref, dr_ref, x_ref, w1_ref, b1_ref, w2_ref, b2_ref,
             w3_ref, b3_ref, out_ref):
    acc = counts_ref[...]
    dacc = dr_ref[...]
    cpieces, dpieces = [], []
    for w in range(_BG // _GPW):
        cpieces.append(jnp.zeros((_NH, _N, _N), jnp.float32))
        cpieces.append(acc[w * _GPW:w * _GPW + _GPW - _NH])
        dpieces.append(jnp.zeros((_NH, 2 * _N), jnp.float32))
        dpieces.append(dacc[w * _GPW:w * _GPW + _GPW - _NH])
    c = acc - jnp.concatenate(cpieces, axis=0)
    dr = dacc - jnp.concatenate(dpieces, axis=0)
    deg = dr[:, :_N] + 1.0
    rdeg = dr[:, _N:] + 1.0
    dinv = lax.rsqrt(deg)

    def cmatvec(v):
        return jnp.sum(c * v[:, :, None], axis=1) + v

    av = cmatvec(dinv)
    s = jnp.sum(dinv * av, axis=1)

    def layer(h, w_ref, b_ref):
        t = lax.dot_general(h, w_ref[...], (((1,), (1,)), ((), ())),
                            preferred_element_type=jnp.float32)
        u = cmatvec(t)
        return s[:, None] * u + _M * b_ref[...]

    h1 = layer(x_ref[...], w1_ref, b1_ref)
    h2 = layer(h1, w2_ref, b2_ref)
    t3 = lax.dot_general(h2, w3_ref[...], (((1,), (1,)), ((), ())),
                         preferred_element_type=jnp.float32)
    tot = s * jnp.sum(rdeg * t3, axis=1) + _M * jnp.sum(b3_ref[...])
    out_ref[...] = (tot * (1.0 / _N)).reshape(1, 1, _BG)


def _tc_chain(counts, dr, x, W1, b1, W2, b2, W3, b3):
    nblk = _G // _BG
    wspec = pl.BlockSpec((_N, _N), lambda i: (0, 0))
    bspec = pl.BlockSpec((1, _N), lambda i: (0, 0))
    out = pl.pallas_call(
        _tc_body,
        grid=(nblk,),
        in_specs=[
            pl.BlockSpec((_BG, _N, _N), lambda i: (i, 0, 0)),
            pl.BlockSpec((_BG, 2 * _N), lambda i: (i, 0)),
            pl.BlockSpec((_BG, _N), lambda i: (i, 0)),
            wspec, bspec, wspec, bspec, wspec, bspec,
        ],
        out_specs=pl.BlockSpec((1, 1, _BG), lambda i: (i, 0, 0)),
        out_shape=jax.ShapeDtypeStruct((nblk, 1, _BG), jnp.float32),
    )(counts, dr, x, W1, b1.reshape(1, _N), W2, b2.reshape(1, _N),
      W3, b3.reshape(1, _N))
    return out.reshape(_G)


def kernel(x, edge_index, W1, b1, W2, b2, W3, b3):
    counts, dr = _sc_hist(edge_index)
    return _tc_chain(counts, dr, x, W1, b1, W2, b2, W3, b3)


# speedup vs baseline: 1266.2513x; 1.0180x over previous
import jax
import jax.numpy as jnp
from jax import lax
from jax.experimental import pallas as pl
from jax.experimental.pallas import tpu as pltpu
from jax.experimental.pallas import tpu_sc as plsc

_G, _N, _E = 2048, 64, 1024
_NC = 2
_NS = 16
_NW = _NC * _NS
_GPW = _G // _NW
_CHUNKS = _E // 16
_UNROLL = 8
_NH = 4
_BG = 4 * _GPW
_M = float(_E + _N)


def _sc_hist_body(edges_hbm, cnt_hbm, dr_hbm, e0, e1, e2, e3, e4, e5, e6, e7,
                  h0, h1, h2, h3, d0, d1, d2, d3,
                  si0, si1, si2, si3, si4, si5, si6, si7,
                  so0, so1, so2, so3, sd0, sd1, sd2, sd3):
    ebufs = (e0, e1, e2, e3, e4, e5, e6, e7)
    hbufs = (h0, h1, h2, h3)
    dbufs = (d0, d1, d2, d3)
    sins = (si0, si1, si2, si3, si4, si5, si6, si7)
    souts = (so0, so1, so2, so3)
    sdrs = (sd0, sd1, sd2, sd3)
    wid = lax.axis_index("s") * _NC + lax.axis_index("c")
    base = wid * _GPW
    ones16 = jnp.ones((16,), jnp.float32)
    zeros16 = jnp.zeros((16,), jnp.float32)

    def zrow(j, carry):
        for hb in hbufs:
            hb[j, pl.ds(0, 16)] = zeros16
            hb[j, pl.ds(16, 16)] = zeros16
            hb[j, pl.ds(32, 16)] = zeros16
            hb[j, pl.ds(48, 16)] = zeros16
        return carry

    lax.fori_loop(0, _N, zrow, 0)
    for db in dbufs:
        for j in range(8):
            db[pl.ds(j * 16, 16)] = zeros16

    for s in range(_UNROLL):
        pltpu.async_copy(edges_hbm.at[base + s], ebufs[s], sins[s])

    def pipe(k, carry):
        g0 = base + _UNROLL * k
        for s in range(_UNROLL):
            g = g0 + s
            eb, sin = ebufs[s], sins[s]
            hb, sout = hbufs[s % _NH], souts[s % _NH]
            db, sdr = dbufs[s % _NH], sdrs[s % _NH]
            pltpu.make_async_copy(edges_hbm.at[g], eb, sin).wait()

            def _waits():
                pltpu.make_async_copy(hb, cnt_hbm.at[g - _NH], sout).wait()
                pltpu.make_async_copy(db, dr_hbm.at[g - _NH], sdr).wait()

            if s < _NH:
                pl.when(k > 0)(_waits)
            else:
                _waits()

            @plsc.parallel_loop(0, _CHUNKS, 1, unroll=8)
            def _(j):
                r = eb[0, pl.ds(j * 16, 16)]
                c = eb[1, pl.ds(j * 16, 16)]
                plsc.addupdate_scatter(hb, [r, c], ones16)
                plsc.addupdate_scatter(db, [c], ones16)
                plsc.addupdate_scatter(db, [r + _N], ones16)

            pltpu.async_copy(hb, cnt_hbm.at[g], sout)
            pltpu.async_copy(db, dr_hbm.at[g], sdr)

            @pl.when(_UNROLL * k + s + _UNROLL < _GPW)
            def _():
                pltpu.async_copy(edges_hbm.at[g + _UNROLL], eb, sin)
        return carry

    lax.fori_loop(0, _GPW // _UNROLL, pipe, 0)

    for j in range(_NH):
        g = base + _GPW - _NH + j
        pltpu.make_async_copy(hbufs[j], cnt_hbm.at[g], souts[j]).wait()
        pltpu.make_async_copy(dbufs[j], dr_hbm.at[g], sdrs[j]).wait()


def _sc_hist(edge_index):
    mesh = plsc.VectorSubcoreMesh(core_axis_name="c", subcore_axis_name="s")
    return pl.kernel(
        _sc_hist_body,
        mesh=mesh,
        out_type=(
            jax.ShapeDtypeStruct((_G, _N, _N), jnp.float32),
            jax.ShapeDtypeStruct((_G, 2 * _N), jnp.float32),
        ),
        scratch_types=(
            [pltpu.VMEM((2, _E), jnp.int32) for _ in range(_UNROLL)]
            + [pltpu.VMEM((_N, _N), jnp.float32) for _ in range(_NH)]
            + [pltpu.VMEM((2 * _N,), jnp.float32) for _ in range(_NH)]
            + [pltpu.SemaphoreType.DMA for _ in range(_UNROLL + 2 * _NH)]
        ),
        compiler_params=pltpu.CompilerParams(needs_layout_passes=False),
    )(edge_index)


def _tc_body(counts_ref, dr_ref, x_ref, w1_ref, b1_ref, w2_ref, b2_ref,
             w3_ref, b3_ref, out_ref):
    acc = counts_ref[...]
    dacc = dr_ref[...]
    cpieces, dpieces = [], []
    for w in range(_BG // _GPW):
        cpieces.append(jnp.zeros((_NH, _N, _N), jnp.float32))
        cpieces.append(acc[w * _GPW:w * _GPW + _GPW - _NH])
        dpieces.append(jnp.zeros((_NH, 2 * _N), jnp.float32))
        dpieces.append(dacc[w * _GPW:w * _GPW + _GPW - _NH])
    c = acc - jnp.concatenate(cpieces, axis=0)
    dr = dacc - jnp.concatenate(dpieces, axis=0)
    deg = dr[:, :_N] + 1.0
    rdeg = dr[:, _N:] + 1.0
    dinv = lax.rsqrt(deg)

    def cmatvec(v):
        return jnp.sum(c * v[:, :, None], axis=1) + v

    av = cmatvec(dinv)
    s = jnp.sum(dinv * av, axis=1)

    def layer(h, w_ref, b_ref):
        t = lax.dot_general(h, w_ref[...], (((1,), (1,)), ((), ())),
                            preferred_element_type=jnp.float32)
        u = cmatvec(t)
        return s[:, None] * u + _M * b_ref[...]

    h1 = layer(x_ref[...], w1_ref, b1_ref)
    h2 = layer(h1, w2_ref, b2_ref)
    t3 = lax.dot_general(h2, w3_ref[...], (((1,), (1,)), ((), ())),
                         preferred_element_type=jnp.float32)
    tot = s * jnp.sum(rdeg * t3, axis=1) + _M * jnp.sum(b3_ref[...])
    out_ref[...] = (tot * (1.0 / _N)).reshape(1, 1, _BG)


def _tc_chain(counts, dr, x, W1, b1, W2, b2, W3, b3):
    nblk = _G // _BG
    wspec = pl.BlockSpec((_N, _N), lambda i: (0, 0))
    bspec = pl.BlockSpec((1, _N), lambda i: (0, 0))
    out = pl.pallas_call(
        _tc_body,
        grid=(nblk,),
        in_specs=[
            pl.BlockSpec((_BG, _N, _N), lambda i: (i, 0, 0)),
            pl.BlockSpec((_BG, 2 * _N), lambda i: (i, 0)),
            pl.BlockSpec((_BG, _N), lambda i: (i, 0)),
            wspec, bspec, wspec, bspec, wspec, bspec,
        ],
        out_specs=pl.BlockSpec((1, 1, _BG), lambda i: (i, 0, 0)),
        out_shape=jax.ShapeDtypeStruct((nblk, 1, _BG), jnp.float32),
    )(counts, dr, x, W1, b1.reshape(1, _N), W2, b2.reshape(1, _N),
      W3, b3.reshape(1, _N))
    return out.reshape(_G)


def kernel(x, edge_index, W1, b1, W2, b2, W3, b3):
    counts, dr = _sc_hist(edge_index)
    return _tc_chain(counts, dr, x, W1, b1, W2, b2, W3, b3)
